# baseline (device time: 166709 ns/iter reference)
import jax
import jax.numpy as jnp
from jax import lax
from jax.experimental import pallas as pl
from jax.experimental.pallas import tpu as pltpu

N_DEV = 4
SQ = 2048
SKV = 2048
D_MODEL = 1024
HQ_PER = 8
DH = 128
WINDOW = 128
KW = 768
SCALE = 0.08838834764831843
CHUNK = SQ // N_DEV
HALF = D_MODEL // 2

CHUNK_ORDER = [0, 3, 1, 2]


def _body(x_ref, wq_ref, k_hbm, v_hbm, wo_ref, out_ref,
          q_ref, kbuf, vbuf, sc_ref, ctx_ref, p_ref,
          send_cw, send_ccw, recv_cw, recv_ccw,
          k_sems, v_sems,
          send_sems_cw, recv_sems_cw, send_sems_ccw, recv_sems_ccw):
    my = lax.axis_index("i")
    left = (my - 1) % N_DEV
    right = (my + 1) % N_DEV

    barrier_sem = pltpu.get_barrier_semaphore()
    for nbr in [left, right]:
        pl.semaphore_signal(barrier_sem, inc=1, device_id=(nbr,),
                            device_id_type=pl.DeviceIdType.MESH)

    def wstart(t):
        c = (my + CHUNK_ORDER[t]) % N_DEV
        return jnp.clip(c * CHUNK - WINDOW, 0, SKV - KW)

    def kv_dma(t):
        slot = t % 2
        start = pl.multiple_of(wstart(t), 128)
        cols = pl.multiple_of(my * (HQ_PER * DH), HQ_PER * DH)
        return (
            pltpu.make_async_copy(
                k_hbm.at[pl.ds(start, KW), pl.ds(cols, HQ_PER * DH)],
                kbuf.at[slot], k_sems.at[slot]),
            pltpu.make_async_copy(
                v_hbm.at[pl.ds(start, KW), pl.ds(cols, HQ_PER * DH)],
                vbuf.at[slot], v_sems.at[slot]),
        )

    def compute_chunk(t):
        slot = t % 2
        c = (my + CHUNK_ORDER[t]) % N_DEV
        qs = c * CHUNK
        start = wstart(t)
        if t + 1 < N_DEV:
            for cp in kv_dma(t + 1):
                cp.start()
        q_ref[:, :] = jnp.dot(x_ref[0, pl.ds(qs, CHUNK), :], wq_ref[:, :],
                              preferred_element_type=jnp.float32)
        qi = qs + lax.broadcasted_iota(jnp.int32, (CHUNK, KW), 0)
        ki = start + lax.broadcasted_iota(jnp.int32, (CHUNK, KW), 1)
        keep = jnp.abs(qi - ki) <= WINDOW
        for cp in kv_dma(t):
            cp.wait()
        for h in range(HQ_PER):
            q = q_ref[:, h * DH:(h + 1) * DH]
            kwin = kbuf[slot, :, h * DH:(h + 1) * DH]
            s = lax.dot_general(
                q, kwin, (((1,), (1,)), ((), ())),
                preferred_element_type=jnp.float32) * SCALE
            sc_ref[:, :] = jnp.where(keep, s, jnp.float32(-1e9))
            s = sc_ref[:, :]
            m = jnp.max(s, axis=-1, keepdims=True)
            w = jnp.exp(s - m)
            w = w / jnp.sum(w, axis=-1, keepdims=True)
            ctx_ref[:, h * DH:(h + 1) * DH] = jnp.dot(
                w, vbuf[slot, :, h * DH:(h + 1) * DH],
                preferred_element_type=jnp.float32)
        p_ref[pl.ds(qs, CHUNK), :] = jnp.dot(
            ctx_ref[:, :], wo_ref[:, :],
            preferred_element_type=jnp.float32).astype(jnp.bfloat16)

    def hop(step):
        cw = pltpu.make_async_remote_copy(
            src_ref=send_cw, dst_ref=recv_cw.at[step],
            send_sem=send_sems_cw.at[step], recv_sem=recv_sems_cw.at[step],
            device_id=(right,), device_id_type=pl.DeviceIdType.MESH,
        )
        ccw = pltpu.make_async_remote_copy(
            src_ref=send_ccw, dst_ref=recv_ccw.at[step],
            send_sem=send_sems_ccw.at[step], recv_sem=recv_sems_ccw.at[step],
            device_id=(left,), device_id_type=pl.DeviceIdType.MESH,
        )
        cw.start()
        ccw.start()
        return cw, ccw

    for cp in kv_dma(0):
        cp.start()
    compute_chunk(0)
    send_cw[:, :] = p_ref[pl.ds(my * CHUNK, CHUNK), :HALF]
    send_ccw[:, :] = p_ref[pl.ds(my * CHUNK, CHUNK), HALF:]
    pl.semaphore_wait(barrier_sem, 2)
    hops = hop(0)

    compute_chunk(1)
    compute_chunk(2)

    for s in range(N_DEV - 1):
        hops[0].wait()
        hops[1].wait()
        cw_idx = (my - s - 1) % N_DEV
        ccw_idx = (my + s + 1) % N_DEV
        acc_cw = (recv_cw[s].astype(jnp.float32)
                  + p_ref[pl.ds(cw_idx * CHUNK, CHUNK), :HALF].astype(
                      jnp.float32))
        acc_ccw = (recv_ccw[s].astype(jnp.float32)
                   + p_ref[pl.ds(ccw_idx * CHUNK, CHUNK), HALF:].astype(
                       jnp.float32))
        send_cw[:, :] = acc_cw.astype(jnp.bfloat16)
        send_ccw[:, :] = acc_ccw.astype(jnp.bfloat16)
        if s < N_DEV - 2:
            hops = hop(s + 1)
            if s == 0:
                compute_chunk(3)
        else:
            out_ref[0, pl.ds(((my + 1) % N_DEV) * CHUNK, CHUNK), :HALF] = acc_cw
            out_ref[0, pl.ds(((my - 1) % N_DEV) * CHUNK, CHUNK), HALF:] = acc_ccw

    for s in range(N_DEV - 1):
        c1, c2 = hop(N_DEV - 1 + s)
        c1.wait()
        c2.wait()
        cw_idx = (my - s) % N_DEV
        ccw_idx = (my + s) % N_DEV
        out_ref[0, pl.ds(cw_idx * CHUNK, CHUNK), :HALF] = (
            recv_cw[N_DEV - 1 + s].astype(jnp.float32))
        out_ref[0, pl.ds(ccw_idx * CHUNK, CHUNK), HALF:] = (
            recv_ccw[N_DEV - 1 + s].astype(jnp.float32))
        if s < N_DEV - 2:
            send_cw[:, :] = recv_cw[N_DEV - 1 + s]
            send_ccw[:, :] = recv_ccw[N_DEV - 1 + s]


def kernel(x, Wq, K_ext, V_ext, Wo):
    k_sh = jnp.reshape(K_ext, (SKV, 32 * DH))
    v_sh = jnp.reshape(V_ext, (SKV, 32 * DH))
    return pl.pallas_call(
        _body,
        out_shape=jax.ShapeDtypeStruct((1, SQ, D_MODEL), jnp.float32),
        in_specs=[
            pl.BlockSpec(memory_space=pltpu.VMEM),
            pl.BlockSpec(memory_space=pltpu.VMEM),
            pl.BlockSpec(memory_space=pltpu.MemorySpace.HBM),
            pl.BlockSpec(memory_space=pltpu.MemorySpace.HBM),
            pl.BlockSpec(memory_space=pltpu.VMEM),
        ],
        out_specs=pl.BlockSpec(memory_space=pltpu.VMEM),
        scratch_shapes=[
            pltpu.VMEM((CHUNK, D_MODEL), jnp.float32),
            pltpu.VMEM((2, KW, HQ_PER * DH), jnp.float32),
            pltpu.VMEM((2, KW, HQ_PER * DH), jnp.float32),
            pltpu.VMEM((CHUNK, KW), jnp.float32),
            pltpu.VMEM((CHUNK, D_MODEL), jnp.float32),
            pltpu.VMEM((SQ, D_MODEL), jnp.bfloat16),
            pltpu.VMEM((CHUNK, HALF), jnp.bfloat16),
            pltpu.VMEM((CHUNK, HALF), jnp.bfloat16),
            pltpu.VMEM((2 * (N_DEV - 1), CHUNK, HALF), jnp.bfloat16),
            pltpu.VMEM((2 * (N_DEV - 1), CHUNK, HALF), jnp.bfloat16),
            pltpu.SemaphoreType.DMA((2,)),
            pltpu.SemaphoreType.DMA((2,)),
            pltpu.SemaphoreType.DMA((2 * (N_DEV - 1),)),
            pltpu.SemaphoreType.DMA((2 * (N_DEV - 1),)),
            pltpu.SemaphoreType.DMA((2 * (N_DEV - 1),)),
            pltpu.SemaphoreType.DMA((2 * (N_DEV - 1),)),
        ],
        compiler_params=pltpu.CompilerParams(
            collective_id=0,
            vmem_limit_bytes=64 * 1024 * 1024,
        ),
    )(x, Wq, k_sh, v_sh, Wo)


# device time: 114332 ns/iter; 1.4581x vs baseline; 1.4581x over previous
import jax
import jax.numpy as jnp
from jax import lax
from jax.experimental import pallas as pl
from jax.experimental.pallas import tpu as pltpu

N_DEV = 4
SQ = 2048
SKV = 2048
D_MODEL = 1024
HQ_PER = 8
DH = 128
WINDOW = 128
KW = 768
SCALE = 0.08838834764831843
CHUNK = SQ // N_DEV
HALF = D_MODEL // 2

CHUNK_ORDER = [0, 3, 1, 2]


def _body(x_hbm, wq_ref, k_ref, v_ref, wo_ref, out_ref,
          xbuf, q_ref, sc_ref, w_ref, ctx_ref, p_ref,
          send_cw, send_ccw, recv_cw, recv_ccw,
          x_sems,
          send_sems_cw, recv_sems_cw, send_sems_ccw, recv_sems_ccw):
    my = lax.axis_index("i")
    left = (my - 1) % N_DEV
    right = (my + 1) % N_DEV

    barrier_sem = pltpu.get_barrier_semaphore()
    for nbr in [left, right]:
        pl.semaphore_signal(barrier_sem, inc=1, device_id=(nbr,),
                            device_id_type=pl.DeviceIdType.MESH)

    def wstart(t):
        c = (my + CHUNK_ORDER[t]) % N_DEV
        return jnp.clip(c * CHUNK - WINDOW, 0, SKV - KW)

    def x_dma(t):
        slot = t % 2
        c = (my + CHUNK_ORDER[t]) % N_DEV
        qs = pl.multiple_of(c * CHUNK, CHUNK)
        return pltpu.make_async_copy(
            x_hbm.at[0, pl.ds(qs, CHUNK), :], xbuf.at[slot],
            x_sems.at[slot])

    def compute_chunk(t):
        slot = t % 2
        c = (my + CHUNK_ORDER[t]) % N_DEV
        qs = c * CHUNK
        start = pl.multiple_of(wstart(t), 128)
        if t + 1 < N_DEV:
            x_dma(t + 1).start()
        x_dma(t).wait()
        q_ref[:, :] = jnp.dot(xbuf[slot], wq_ref[:, :],
                              preferred_element_type=jnp.float32)
        qi = qs + lax.broadcasted_iota(jnp.int32, (CHUNK, KW), 0)
        ki = start + lax.broadcasted_iota(jnp.int32, (CHUNK, KW), 1)
        keep = jnp.abs(qi - ki) <= WINDOW
        for h in range(HQ_PER):
            q = q_ref[:, h * DH:(h + 1) * DH]
            kwin = k_ref[h, pl.ds(start, KW), :]
            s = lax.dot_general(
                q, kwin, (((1,), (1,)), ((), ())),
                preferred_element_type=jnp.float32) * SCALE
            sc_ref[:, :] = jnp.where(keep, s, jnp.float32(-1e9))
            s = sc_ref[:, :]
            m = jnp.max(s, axis=-1, keepdims=True)
            w_ref[:, :] = jnp.exp(s - m)
            w = w_ref[:, :] / jnp.sum(w_ref[:, :], axis=-1, keepdims=True)
            ctx_ref[:, h * DH:(h + 1) * DH] = jnp.dot(
                w, v_ref[h, pl.ds(start, KW), :],
                preferred_element_type=jnp.float32)
        p_ref[pl.ds(qs, CHUNK), :] = jnp.dot(
            ctx_ref[:, :], wo_ref[:, :],
            preferred_element_type=jnp.float32).astype(jnp.bfloat16)

    def hop(step):
        cw = pltpu.make_async_remote_copy(
            src_ref=send_cw, dst_ref=recv_cw.at[step],
            send_sem=send_sems_cw.at[step], recv_sem=recv_sems_cw.at[step],
            device_id=(right,), device_id_type=pl.DeviceIdType.MESH,
        )
        ccw = pltpu.make_async_remote_copy(
            src_ref=send_ccw, dst_ref=recv_ccw.at[step],
            send_sem=send_sems_ccw.at[step], recv_sem=recv_sems_ccw.at[step],
            device_id=(left,), device_id_type=pl.DeviceIdType.MESH,
        )
        cw.start()
        ccw.start()
        return cw, ccw

    x_dma(0).start()
    compute_chunk(0)
    send_cw[:, :] = p_ref[pl.ds(my * CHUNK, CHUNK), :HALF]
    send_ccw[:, :] = p_ref[pl.ds(my * CHUNK, CHUNK), HALF:]
    pl.semaphore_wait(barrier_sem, 2)
    hops = hop(0)

    compute_chunk(1)
    compute_chunk(2)

    for s in range(N_DEV - 1):
        hops[0].wait()
        hops[1].wait()
        cw_idx = (my - s - 1) % N_DEV
        ccw_idx = (my + s + 1) % N_DEV
        acc_cw = (recv_cw[s].astype(jnp.float32)
                  + p_ref[pl.ds(cw_idx * CHUNK, CHUNK), :HALF].astype(
                      jnp.float32))
        acc_ccw = (recv_ccw[s].astype(jnp.float32)
                   + p_ref[pl.ds(ccw_idx * CHUNK, CHUNK), HALF:].astype(
                       jnp.float32))
        send_cw[:, :] = acc_cw.astype(jnp.bfloat16)
        send_ccw[:, :] = acc_ccw.astype(jnp.bfloat16)
        if s < N_DEV - 2:
            hops = hop(s + 1)
            if s == 0:
                compute_chunk(3)
        else:
            out_ref[0, pl.ds(((my + 1) % N_DEV) * CHUNK, CHUNK), :HALF] = acc_cw
            out_ref[0, pl.ds(((my - 1) % N_DEV) * CHUNK, CHUNK), HALF:] = acc_ccw

    for s in range(N_DEV - 1):
        c1, c2 = hop(N_DEV - 1 + s)
        c1.wait()
        c2.wait()
        cw_idx = (my - s) % N_DEV
        ccw_idx = (my + s) % N_DEV
        out_ref[0, pl.ds(cw_idx * CHUNK, CHUNK), :HALF] = (
            recv_cw[N_DEV - 1 + s].astype(jnp.float32))
        out_ref[0, pl.ds(ccw_idx * CHUNK, CHUNK), HALF:] = (
            recv_ccw[N_DEV - 1 + s].astype(jnp.float32))
        if s < N_DEV - 2:
            send_cw[:, :] = recv_cw[N_DEV - 1 + s]
            send_ccw[:, :] = recv_ccw[N_DEV - 1 + s]


def kernel(x, Wq, K_ext, V_ext, Wo):
    my = lax.axis_index("i")
    k_sh = jnp.transpose(
        lax.dynamic_slice_in_dim(K_ext[0], my * HQ_PER, HQ_PER, axis=1),
        (1, 0, 2))
    v_sh = jnp.transpose(
        lax.dynamic_slice_in_dim(V_ext[0], my * HQ_PER, HQ_PER, axis=1),
        (1, 0, 2))
    return pl.pallas_call(
        _body,
        out_shape=jax.ShapeDtypeStruct((1, SQ, D_MODEL), jnp.float32),
        in_specs=[
            pl.BlockSpec(memory_space=pltpu.MemorySpace.HBM),
            pl.BlockSpec(memory_space=pltpu.VMEM),
            pl.BlockSpec(memory_space=pltpu.VMEM),
            pl.BlockSpec(memory_space=pltpu.VMEM),
            pl.BlockSpec(memory_space=pltpu.VMEM),
        ],
        out_specs=pl.BlockSpec(memory_space=pltpu.VMEM),
        scratch_shapes=[
            pltpu.VMEM((2, CHUNK, D_MODEL), jnp.float32),
            pltpu.VMEM((CHUNK, D_MODEL), jnp.float32),
            pltpu.VMEM((CHUNK, KW), jnp.float32),
            pltpu.VMEM((CHUNK, KW), jnp.float32),
            pltpu.VMEM((CHUNK, D_MODEL), jnp.float32),
            pltpu.VMEM((SQ, D_MODEL), jnp.bfloat16),
            pltpu.VMEM((CHUNK, HALF), jnp.bfloat16),
            pltpu.VMEM((CHUNK, HALF), jnp.bfloat16),
            pltpu.VMEM((2 * (N_DEV - 1), CHUNK, HALF), jnp.bfloat16),
            pltpu.VMEM((2 * (N_DEV - 1), CHUNK, HALF), jnp.bfloat16),
            pltpu.SemaphoreType.DMA((2,)),
            pltpu.SemaphoreType.DMA((2 * (N_DEV - 1),)),
            pltpu.SemaphoreType.DMA((2 * (N_DEV - 1),)),
            pltpu.SemaphoreType.DMA((2 * (N_DEV - 1),)),
            pltpu.SemaphoreType.DMA((2 * (N_DEV - 1),)),
        ],
        compiler_params=pltpu.CompilerParams(
            collective_id=0,
            vmem_limit_bytes=64 * 1024 * 1024,
        ),
    )(x, Wq, k_sh, v_sh, Wo)


# device time: 111307 ns/iter; 1.4977x vs baseline; 1.0272x over previous
import jax
import jax.numpy as jnp
from jax import lax
from jax.experimental import pallas as pl
from jax.experimental.pallas import tpu as pltpu

N_DEV = 4
SQ = 2048
SKV = 2048
D_MODEL = 1024
HQ_PER = 8
DH = 128
WINDOW = 128
KW = 768
SCALE = 0.08838834764831843
CHUNK = SQ // N_DEV
HALF = D_MODEL // 2

CHUNK_ORDER = [0, 3, 1, 2]


def _body(x_hbm, wq_ref, k_ref, v_ref, wo_ref, out_ref,
          xbuf, q_ref, kwin_ref, vwin_ref, sc_ref, w_ref, ctx_ref, p_ref,
          send_cw, send_ccw, recv_cw, recv_ccw,
          x_sems,
          send_sems_cw, recv_sems_cw, send_sems_ccw, recv_sems_ccw):
    my = lax.axis_index("i")
    left = (my - 1) % N_DEV
    right = (my + 1) % N_DEV

    barrier_sem = pltpu.get_barrier_semaphore()
    for nbr in [left, right]:
        pl.semaphore_signal(barrier_sem, inc=1, device_id=(nbr,),
                            device_id_type=pl.DeviceIdType.MESH)

    def wstart(t):
        c = (my + CHUNK_ORDER[t]) % N_DEV
        return jnp.clip(c * CHUNK - WINDOW, 0, SKV - KW)

    def x_dma(t):
        slot = t % 2
        c = (my + CHUNK_ORDER[t]) % N_DEV
        qs = pl.multiple_of(c * CHUNK, CHUNK)
        return pltpu.make_async_copy(
            x_hbm.at[0, pl.ds(qs, CHUNK), :], xbuf.at[slot],
            x_sems.at[slot])

    def compute_chunk(t):
        slot = t % 2
        c = (my + CHUNK_ORDER[t]) % N_DEV
        qs = c * CHUNK
        start = wstart(t)
        if t + 1 < N_DEV:
            x_dma(t + 1).start()
        x_dma(t).wait()
        q_ref[:, :] = jnp.dot(xbuf[slot], wq_ref[:, :],
                              preferred_element_type=jnp.float32)
        for cc in range(N_DEV):
            st = min(max(cc * CHUNK - WINDOW, 0), SKV - KW)

            @pl.when(c == cc)
            def _(st=st):
                kwin_ref[:, :, :] = k_ref[:, st:st + KW, :]
                vwin_ref[:, :, :] = v_ref[:, st:st + KW, :]

        qi = qs + lax.broadcasted_iota(jnp.int32, (CHUNK, KW), 0)
        ki = start + lax.broadcasted_iota(jnp.int32, (CHUNK, KW), 1)
        keep = jnp.abs(qi - ki) <= WINDOW
        for h in range(HQ_PER):
            q = q_ref[:, h * DH:(h + 1) * DH]
            s = lax.dot_general(
                q, kwin_ref[h], (((1,), (1,)), ((), ())),
                preferred_element_type=jnp.float32) * SCALE
            sc_ref[:, :] = jnp.where(keep, s, jnp.float32(-1e9))
            s = sc_ref[:, :]
            m = jnp.max(s, axis=-1, keepdims=True)
            w_ref[:, :] = jnp.exp(s - m)
            w = w_ref[:, :] / jnp.sum(w_ref[:, :], axis=-1, keepdims=True)
            ctx_ref[:, h * DH:(h + 1) * DH] = jnp.dot(
                w, vwin_ref[h], preferred_element_type=jnp.float32)
        p_ref[pl.ds(pl.multiple_of(qs, CHUNK), CHUNK), :] = jnp.dot(
            ctx_ref[:, :], wo_ref[:, :],
            preferred_element_type=jnp.float32).astype(jnp.bfloat16)

    def hop(step):
        cw = pltpu.make_async_remote_copy(
            src_ref=send_cw, dst_ref=recv_cw.at[step],
            send_sem=send_sems_cw.at[step], recv_sem=recv_sems_cw.at[step],
            device_id=(right,), device_id_type=pl.DeviceIdType.MESH,
        )
        ccw = pltpu.make_async_remote_copy(
            src_ref=send_ccw, dst_ref=recv_ccw.at[step],
            send_sem=send_sems_ccw.at[step], recv_sem=recv_sems_ccw.at[step],
            device_id=(left,), device_id_type=pl.DeviceIdType.MESH,
        )
        cw.start()
        ccw.start()
        return cw, ccw

    x_dma(0).start()
    compute_chunk(0)
    send_cw[:, :] = p_ref[pl.ds(my * CHUNK, CHUNK), :HALF]
    send_ccw[:, :] = p_ref[pl.ds(my * CHUNK, CHUNK), HALF:]
    pl.semaphore_wait(barrier_sem, 2)
    hops = hop(0)

    compute_chunk(1)
    compute_chunk(2)

    for s in range(N_DEV - 1):
        hops[0].wait()
        hops[1].wait()
        cw_idx = (my - s - 1) % N_DEV
        ccw_idx = (my + s + 1) % N_DEV
        acc_cw = (recv_cw[s].astype(jnp.float32)
                  + p_ref[pl.ds(cw_idx * CHUNK, CHUNK), :HALF].astype(
                      jnp.float32))
        acc_ccw = (recv_ccw[s].astype(jnp.float32)
                   + p_ref[pl.ds(ccw_idx * CHUNK, CHUNK), HALF:].astype(
                       jnp.float32))
        send_cw[:, :] = acc_cw.astype(jnp.bfloat16)
        send_ccw[:, :] = acc_ccw.astype(jnp.bfloat16)
        if s < N_DEV - 2:
            hops = hop(s + 1)
            if s == 0:
                compute_chunk(3)
        else:
            out_ref[0, pl.ds(((my + 1) % N_DEV) * CHUNK, CHUNK), :HALF] = acc_cw
            out_ref[0, pl.ds(((my - 1) % N_DEV) * CHUNK, CHUNK), HALF:] = acc_ccw

    for s in range(N_DEV - 1):
        c1, c2 = hop(N_DEV - 1 + s)
        c1.wait()
        c2.wait()
        cw_idx = (my - s) % N_DEV
        ccw_idx = (my + s) % N_DEV
        out_ref[0, pl.ds(cw_idx * CHUNK, CHUNK), :HALF] = (
            recv_cw[N_DEV - 1 + s].astype(jnp.float32))
        out_ref[0, pl.ds(ccw_idx * CHUNK, CHUNK), HALF:] = (
            recv_ccw[N_DEV - 1 + s].astype(jnp.float32))
        if s < N_DEV - 2:
            send_cw[:, :] = recv_cw[N_DEV - 1 + s]
            send_ccw[:, :] = recv_ccw[N_DEV - 1 + s]


def kernel(x, Wq, K_ext, V_ext, Wo):
    my = lax.axis_index("i")
    k_sh = jnp.transpose(
        lax.dynamic_slice_in_dim(K_ext[0], my * HQ_PER, HQ_PER, axis=1),
        (1, 0, 2))
    v_sh = jnp.transpose(
        lax.dynamic_slice_in_dim(V_ext[0], my * HQ_PER, HQ_PER, axis=1),
        (1, 0, 2))
    return pl.pallas_call(
        _body,
        out_shape=jax.ShapeDtypeStruct((1, SQ, D_MODEL), jnp.float32),
        in_specs=[
            pl.BlockSpec(memory_space=pltpu.MemorySpace.HBM),
            pl.BlockSpec(memory_space=pltpu.VMEM),
            pl.BlockSpec(memory_space=pltpu.VMEM),
            pl.BlockSpec(memory_space=pltpu.VMEM),
            pl.BlockSpec(memory_space=pltpu.VMEM),
        ],
        out_specs=pl.BlockSpec(memory_space=pltpu.VMEM),
        scratch_shapes=[
            pltpu.VMEM((2, CHUNK, D_MODEL), jnp.float32),
            pltpu.VMEM((CHUNK, D_MODEL), jnp.float32),
            pltpu.VMEM((HQ_PER, KW, DH), jnp.float32),
            pltpu.VMEM((HQ_PER, KW, DH), jnp.float32),
            pltpu.VMEM((CHUNK, KW), jnp.float32),
            pltpu.VMEM((CHUNK, KW), jnp.float32),
            pltpu.VMEM((CHUNK, D_MODEL), jnp.float32),
            pltpu.VMEM((SQ, D_MODEL), jnp.bfloat16),
            pltpu.VMEM((CHUNK, HALF), jnp.bfloat16),
            pltpu.VMEM((CHUNK, HALF), jnp.bfloat16),
            pltpu.VMEM((2 * (N_DEV - 1), CHUNK, HALF), jnp.bfloat16),
            pltpu.VMEM((2 * (N_DEV - 1), CHUNK, HALF), jnp.bfloat16),
            pltpu.SemaphoreType.DMA((2,)),
            pltpu.SemaphoreType.DMA((2 * (N_DEV - 1),)),
            pltpu.SemaphoreType.DMA((2 * (N_DEV - 1),)),
            pltpu.SemaphoreType.DMA((2 * (N_DEV - 1),)),
            pltpu.SemaphoreType.DMA((2 * (N_DEV - 1),)),
        ],
        compiler_params=pltpu.CompilerParams(
            collective_id=0,
            vmem_limit_bytes=64 * 1024 * 1024,
        ),
    )(x, Wq, k_sh, v_sh, Wo)


# device time: 110876 ns/iter; 1.5036x vs baseline; 1.0039x over previous
import jax
import jax.numpy as jnp
from jax import lax
from jax.experimental import pallas as pl
from jax.experimental.pallas import tpu as pltpu

N_DEV = 4
SQ = 2048
SKV = 2048
D_MODEL = 1024
HQ_PER = 8
DH = 128
WINDOW = 128
KW = 768
SCALE = 0.08838834764831843
CHUNK = SQ // N_DEV
HALF = D_MODEL // 2

CHUNK_ORDER = [0, 3, 1, 2]


def _body(x_hbm, wq_ref, k_ref, v_ref, wo_ref, out_ref,
          xbuf, q_ref, kwin_ref, vwin_ref, sc_ref, w_ref, ctx_ref, p_ref,
          send_cw, send_ccw, recv_cw, recv_ccw,
          x_sems,
          send_sems_cw, recv_sems_cw, send_sems_ccw, recv_sems_ccw):
    my = lax.axis_index("i")
    left = (my - 1) % N_DEV
    right = (my + 1) % N_DEV

    barrier_sem = pltpu.get_barrier_semaphore()
    for nbr in [left, right]:
        pl.semaphore_signal(barrier_sem, inc=1, device_id=(nbr,),
                            device_id_type=pl.DeviceIdType.MESH)

    def wstart(t):
        c = (my + CHUNK_ORDER[t]) % N_DEV
        return jnp.clip(c * CHUNK - WINDOW, 0, SKV - KW)

    def x_dma(t):
        slot = t % 2
        c = (my + CHUNK_ORDER[t]) % N_DEV
        qs = pl.multiple_of(c * CHUNK, CHUNK)
        return pltpu.make_async_copy(
            x_hbm.at[0, pl.ds(qs, CHUNK), :], xbuf.at[slot],
            x_sems.at[slot])

    def compute_chunk(t):
        slot = t % 2
        c = (my + CHUNK_ORDER[t]) % N_DEV
        qs = c * CHUNK
        start = wstart(t)
        if t + 1 < N_DEV:
            x_dma(t + 1).start()
        x_dma(t).wait()
        q_ref[:, :] = jnp.dot(xbuf[slot], wq_ref[:, :],
                              preferred_element_type=jnp.float32)
        for cc in range(N_DEV):
            st = min(max(cc * CHUNK - WINDOW, 0), SKV - KW)

            @pl.when(c == cc)
            def _(st=st):
                kwin_ref[:, :, :] = k_ref[:, st:st + KW, :]
                vwin_ref[:, :, :] = v_ref[:, st:st + KW, :]

        qi = qs + lax.broadcasted_iota(jnp.int32, (CHUNK, KW), 0)
        ki = start + lax.broadcasted_iota(jnp.int32, (CHUNK, KW), 1)
        keep = jnp.abs(qi - ki) <= WINDOW
        for h in range(HQ_PER):
            q = q_ref[:, h * DH:(h + 1) * DH]
            s = lax.dot_general(
                q, kwin_ref[h], (((1,), (1,)), ((), ())),
                preferred_element_type=jnp.float32) * SCALE
            sc_ref[:, :] = jnp.where(keep, s, jnp.float32(-1e9))
            s = sc_ref[:, :]
            m = jnp.max(s, axis=-1, keepdims=True)
            w_ref[:, :] = jnp.exp(s - m)
            w = w_ref[:, :] / jnp.sum(w_ref[:, :], axis=-1, keepdims=True)
            ctx_ref[:, h * DH:(h + 1) * DH] = jnp.dot(
                w, vwin_ref[h], preferred_element_type=jnp.float32)
        p_ref[pl.ds(pl.multiple_of(qs, CHUNK), CHUNK), :] = jnp.dot(
            ctx_ref[:, :], wo_ref[:, :],
            preferred_element_type=jnp.float32).astype(jnp.bfloat16)

    def hop_cw(step):
        cw = pltpu.make_async_remote_copy(
            src_ref=send_cw, dst_ref=recv_cw.at[step],
            send_sem=send_sems_cw.at[step], recv_sem=recv_sems_cw.at[step],
            device_id=(right,), device_id_type=pl.DeviceIdType.MESH,
        )
        cw.start()
        return cw

    def hop_ccw(step):
        ccw = pltpu.make_async_remote_copy(
            src_ref=send_ccw, dst_ref=recv_ccw.at[step],
            send_sem=send_sems_ccw.at[step], recv_sem=recv_sems_ccw.at[step],
            device_id=(left,), device_id_type=pl.DeviceIdType.MESH,
        )
        ccw.start()
        return ccw

    x_dma(0).start()
    compute_chunk(0)
    send_cw[:, :] = p_ref[pl.ds(my * CHUNK, CHUNK), :HALF]
    send_ccw[:, :] = p_ref[pl.ds(my * CHUNK, CHUNK), HALF:]
    pl.semaphore_wait(barrier_sem, 2)
    hcw = hop_cw(0)
    hccw = hop_ccw(0)

    compute_chunk(1)
    compute_chunk(2)

    acc_cw = acc_ccw = None
    for s in range(N_DEV - 1):
        hcw.wait()
        cw_idx = (my - s - 1) % N_DEV
        acc_cw = (recv_cw[s].astype(jnp.float32)
                  + p_ref[pl.ds(cw_idx * CHUNK, CHUNK), :HALF].astype(
                      jnp.float32))
        send_cw[:, :] = acc_cw.astype(jnp.bfloat16)
        if s < N_DEV - 2:
            hcw = hop_cw(s + 1)
        hccw.wait()
        ccw_idx = (my + s + 1) % N_DEV
        acc_ccw = (recv_ccw[s].astype(jnp.float32)
                   + p_ref[pl.ds(ccw_idx * CHUNK, CHUNK), HALF:].astype(
                       jnp.float32))
        send_ccw[:, :] = acc_ccw.astype(jnp.bfloat16)
        if s < N_DEV - 2:
            hccw = hop_ccw(s + 1)
            if s == 0:
                compute_chunk(3)

    hcw = hop_cw(N_DEV - 1)
    hccw = hop_ccw(N_DEV - 1)
    out_ref[0, pl.ds(((my + 1) % N_DEV) * CHUNK, CHUNK), :HALF] = acc_cw
    out_ref[0, pl.ds(((my - 1) % N_DEV) * CHUNK, CHUNK), HALF:] = acc_ccw
    for s in range(N_DEV - 1):
        hcw.wait()
        if s < N_DEV - 2:
            send_cw[:, :] = recv_cw[N_DEV - 1 + s]
            hcw = hop_cw(N_DEV + s)
        hccw.wait()
        if s < N_DEV - 2:
            send_ccw[:, :] = recv_ccw[N_DEV - 1 + s]
            hccw = hop_ccw(N_DEV + s)
        cw_idx = (my - s) % N_DEV
        ccw_idx = (my + s) % N_DEV
        out_ref[0, pl.ds(cw_idx * CHUNK, CHUNK), :HALF] = (
            recv_cw[N_DEV - 1 + s].astype(jnp.float32))
        out_ref[0, pl.ds(ccw_idx * CHUNK, CHUNK), HALF:] = (
            recv_ccw[N_DEV - 1 + s].astype(jnp.float32))


def kernel(x, Wq, K_ext, V_ext, Wo):
    my = lax.axis_index("i")
    k_sh = jnp.transpose(
        lax.dynamic_slice_in_dim(K_ext[0], my * HQ_PER, HQ_PER, axis=1),
        (1, 0, 2))
    v_sh = jnp.transpose(
        lax.dynamic_slice_in_dim(V_ext[0], my * HQ_PER, HQ_PER, axis=1),
        (1, 0, 2))
    return pl.pallas_call(
        _body,
        out_shape=jax.ShapeDtypeStruct((1, SQ, D_MODEL), jnp.float32),
        in_specs=[
            pl.BlockSpec(memory_space=pltpu.MemorySpace.HBM),
            pl.BlockSpec(memory_space=pltpu.VMEM),
            pl.BlockSpec(memory_space=pltpu.VMEM),
            pl.BlockSpec(memory_space=pltpu.VMEM),
            pl.BlockSpec(memory_space=pltpu.VMEM),
        ],
        out_specs=pl.BlockSpec(memory_space=pltpu.VMEM),
        scratch_shapes=[
            pltpu.VMEM((2, CHUNK, D_MODEL), jnp.float32),
            pltpu.VMEM((CHUNK, D_MODEL), jnp.float32),
            pltpu.VMEM((HQ_PER, KW, DH), jnp.float32),
            pltpu.VMEM((HQ_PER, KW, DH), jnp.float32),
            pltpu.VMEM((CHUNK, KW), jnp.float32),
            pltpu.VMEM((CHUNK, KW), jnp.float32),
            pltpu.VMEM((CHUNK, D_MODEL), jnp.float32),
            pltpu.VMEM((SQ, D_MODEL), jnp.bfloat16),
            pltpu.VMEM((CHUNK, HALF), jnp.bfloat16),
            pltpu.VMEM((CHUNK, HALF), jnp.bfloat16),
            pltpu.VMEM((2 * (N_DEV - 1), CHUNK, HALF), jnp.bfloat16),
            pltpu.VMEM((2 * (N_DEV - 1), CHUNK, HALF), jnp.bfloat16),
            pltpu.SemaphoreType.DMA((2,)),
            pltpu.SemaphoreType.DMA((2 * (N_DEV - 1),)),
            pltpu.SemaphoreType.DMA((2 * (N_DEV - 1),)),
            pltpu.SemaphoreType.DMA((2 * (N_DEV - 1),)),
            pltpu.SemaphoreType.DMA((2 * (N_DEV - 1),)),
        ],
        compiler_params=pltpu.CompilerParams(
            collective_id=0,
            vmem_limit_bytes=64 * 1024 * 1024,
        ),
    )(x, Wq, k_sh, v_sh, Wo)
